# baseline (speedup 1.0000x reference)
import jax
import jax.numpy as jnp
from jax.experimental import pallas as pl
from jax.experimental.pallas import tpu as pltpu

B, G, NGO, NKE, NN, GRID, NT, DC = 32, 10000, 2000, 500, 2, 2, 50, 256
F32 = jnp.float32
BF16 = jnp.bfloat16
MBLK = 400
NSTEP = NGO // MBLK
NBALL = NGO + 3 * NKE


def _dotT(a, b):
    return jax.lax.dot_general(a, b, (((1,), (1,)), ((), ())),
                               preferred_element_type=F32)


def _fk2(x0, x1, cs_ref, base):
    def c(ci, j, i, g):
        return cs_ref[0, base + ci * 8 + j * 4 + i * 2 + g]
    c10, s10 = jnp.cos(x0), jnp.sin(x0)
    c11, s11 = jnp.cos(x1), jnp.sin(x1)
    c20 = 2.0 * c10 * c10 - 1.0
    s20 = 2.0 * s10 * c10
    c21 = 2.0 * c11 * c11 - 1.0
    s21 = 2.0 * s11 * c11
    y0 = (c10 * c(0, 0, 0, 0) + c20 * c(0, 0, 0, 1) + c11 * c(0, 0, 1, 0)
          + c21 * c(0, 0, 1, 1) + s10 * c(1, 0, 0, 0) + s20 * c(1, 0, 0, 1)
          + s11 * c(1, 0, 1, 0) + s21 * c(1, 0, 1, 1))
    y1 = (c10 * c(0, 1, 0, 0) + c20 * c(0, 1, 0, 1) + c11 * c(0, 1, 1, 0)
          + c21 * c(0, 1, 1, 1) + s10 * c(1, 1, 0, 0) + s20 * c(1, 1, 0, 1)
          + s11 * c(1, 1, 1, 0) + s21 * c(1, 1, 1, 1))
    return y0, y1


def _fourier32(x, p, bias):
    j = p.shape[1] // 4
    cx, sx = jnp.cos(x), jnp.sin(x)
    c2, s2 = 2.0 * cx * cx - 1.0, 2.0 * sx * cx
    y = (jnp.dot(cx, p[:, 0:j], preferred_element_type=F32)
         + jnp.dot(c2, p[:, j:2 * j], preferred_element_type=F32)
         + jnp.dot(sx, p[:, 2 * j:3 * j], preferred_element_type=F32)
         + jnp.dot(s2, p[:, 3 * j:4 * j], preferred_element_type=F32))
    return y + bias


def _body(gene_hbm, adj_ref, gam_hbm, bet_hbm, wsp_hbm, goke_hbm, keke_hbm,
          nbc_hbm, comp_hbm, wdrug_hbm, wbio_hbm, tis_hbm, cb_hbm, cd_hbm,
          cp_hbm, bbio_hbm, bdrug_hbm, b1b_hbm, d1b_hbm, cs_ref,
          pred_ref, state_ref,
          gp_s, wsp_s, zt_ref, h_ref, goke_s, keke_s, nbc_s, comp_s, wdrug_s,
          wbio_s, tis_s, cb_s, cd_s, cp_s, bbio_s, bdrug_s, b1b_s, d1b_s,
          nbt_s, p_s, m_s, wb_s, dk_s, sp_ref, sem):
    i = pl.program_id(0)
    sc = lambda k: cs_ref[0, k]
    copies = [
        (gene_hbm, gp_s.at[0:B, :]), (gam_hbm, gp_s.at[B:B + 1, :]),
        (bet_hbm, gp_s.at[B + 1:B + 2, :]), (wsp_hbm, wsp_s),
        (goke_hbm, goke_s), (keke_hbm, keke_s), (nbc_hbm, nbc_s),
        (comp_hbm, comp_s), (wdrug_hbm, wdrug_s), (wbio_hbm, wbio_s),
        (tis_hbm, tis_s), (cb_hbm, cb_s), (cd_hbm, cd_s), (cp_hbm, cp_s),
        (bbio_hbm, bbio_s), (bdrug_hbm, bdrug_s), (b1b_hbm, b1b_s),
        (d1b_hbm, d1b_s),
    ]

    @pl.when(i == 0)
    def _build_zt():
        for k, (src, dst) in enumerate(copies):
            pltpu.make_async_copy(src, dst, sem.at[k]).start()
        for k in range(4):
            src, dst = copies[k]
            pltpu.make_async_copy(src, dst, sem.at[k]).wait()
        g = gp_s[0:B, :]
        t0 = jnp.tanh(g * sc(0) + sc(2))
        t1 = jnp.tanh(g * sc(1) + sc(3))
        s1 = jnp.sum(t0, axis=0, keepdims=True) + jnp.sum(t1, axis=0, keepdims=True)
        s2 = jnp.sum(t0 * t0, axis=0, keepdims=True) + jnp.sum(t1 * t1, axis=0, keepdims=True)
        mean = s1 * (1.0 / (2 * B))
        var = s2 * (1.0 / (2 * B)) - mean * mean
        inv = jax.lax.rsqrt(var + 1e-5) * gp_s[B:B + 1, :]
        bet = gp_s[B + 1:B + 2, :]
        zn0 = ((t0 - mean) * inv + bet).astype(BF16)
        zn1 = ((t1 - mean) * inv + bet).astype(BF16)
        zt_ref[...] = jnp.zeros((G, 128), BF16)
        zt_ref[:, 0:32] = jnp.transpose(zn0)
        zt_ref[:, 32:64] = jnp.transpose(zn1)
        zt_ref[:, 64:65] = jnp.ones((G, 1), BF16)
        sp_ref[...] = jnp.dot(wsp_s[:, 0:G].astype(BF16), zt_ref[...],
                              preferred_element_type=F32)

    h_ref[pl.ds(i * MBLK, MBLK), :] = jnp.dot(
        adj_ref[...].astype(BF16), zt_ref[...], preferred_element_type=F32)

    @pl.when(i == 1)
    def _prep_small():
        for k in range(6, len(copies)):
            src, dst = copies[k]
            pltpu.make_async_copy(src, dst, sem.at[k]).wait()
        nbt_s[...] = jnp.transpose(nbc_s[...])
        li = jax.lax.broadcasted_iota(jnp.int32, (B, 64), 1)
        si = jax.lax.broadcasted_iota(jnp.int32, (B, 64), 0)
        e0 = (li == 2 * si).astype(F32)
        e1 = (li == 2 * si + 1).astype(F32)
        p_s[:, 0:16] = _dotT(e0, cb_s[0:16, :])
        p_s[:, 16:32] = _dotT(e1, cb_s[0:16, :])
        p_s[:, 32:48] = _dotT(e0, cb_s[16:32, :])
        p_s[:, 48:64] = _dotT(e1, cb_s[16:32, :])
        p_s[:, 64:80] = _dotT(e0, cd_s[0:16, :])
        p_s[:, 80:96] = _dotT(e1, cd_s[0:16, :])
        p_s[:, 96:112] = _dotT(e0, cd_s[16:32, :])
        p_s[:, 112:128] = _dotT(e1, cd_s[16:32, :])
        p_s[:, 128:129] = _dotT(e0, cp_s[0:1, :])
        p_s[:, 129:130] = _dotT(e1, cp_s[0:1, :])
        p_s[:, 130:131] = _dotT(e0, cp_s[1:2, :])
        p_s[:, 131:132] = _dotT(e1, cp_s[1:2, :])
        tis64 = jnp.concatenate(
            [tis_s[...], jnp.full((1, 64 - NT), -1, jnp.int32)], axis=1)
        kidx = jax.lax.broadcasted_iota(jnp.int32, (NKE, 64), 0)
        m_s[...] = (kidx == tis64).astype(F32)
        wb_s[...] = jnp.concatenate(
            [wbio_s[...], jnp.zeros((B, 64 - NT), F32)], axis=1)

    @pl.when(i == 2)
    def _prep_deg():
        for k in (4, 5):
            src, dst = copies[k]
            pltpu.make_async_copy(src, dst, sem.at[k]).wait()
        dk_s[:, 0:NKE] = _dotT(jnp.ones((1, NGO), F32), goke_s[...]) + 1e-8
        dk_s[:, 512:512 + NKE] = (_dotT(jnp.ones((1, NKE), F32), keke_s[...])
                                  + 1e-8)

    @pl.when(i == NSTEP - 1)
    def _tail():
        ht = jnp.transpose(h_ref[...])
        degT = ht[64:65, :] + 1e-8
        x0 = ht[0:32, :] / degT
        x1 = ht[32:64, :] / degT
        y0, y1 = _fk2(x0, x1, cs_ref, 32)
        y0 = y0 + nbt_s[0:1, 0:NGO]
        y1 = y1 + nbt_s[1:2, 0:NGO]
        e = y0 * sc(4) + y1 * sc(5) + sc(6)
        gf0 = jnp.tanh(e * sc(7) + sc(9))
        gf1 = jnp.tanh(e * sc(8) + sc(10))
        gost = gf0 * sc(11) + gf1 * sc(12)
        wspgo = wsp_s[:, G:G + NGO]
        sp_go = _dotT(wspgo, gost) + sc(13) * jnp.sum(wspgo)

        a = goke_s[...]
        dkg = dk_s[:, 0:NKE]
        kh0 = _dotT(gf0, a) / dkg
        kh1 = _dotT(gf1, a) / dkg
        y0, y1 = _fk2(kh0, kh1, cs_ref, 48)
        k0 = y0 + nbt_s[0:1, NGO:NGO + NKE]
        k1 = y1 + nbt_s[1:2, NGO:NGO + NKE]

        kk = keke_s[...]
        dkk = dk_s[:, 512:512 + NKE]
        x0 = _dotT(k0, kk) / dkk
        x1 = _dotT(k1, kk) / dkk
        y0, y1 = _fk2(x0, x1, cs_ref, 64)
        k0 = y0 + nbt_s[0:1, NGO + NKE:NGO + 2 * NKE]
        k1 = y1 + nbt_s[1:2, NGO + NKE:NGO + 2 * NKE]
        x0 = _dotT(k0, kk) / dkk
        x1 = _dotT(k1, kk) / dkk
        y0, y1 = _fk2(x0, x1, cs_ref, 80)
        k0 = y0 + nbt_s[0:1, NGO + 2 * NKE:NGO + 3 * NKE]
        k1 = y1 + nbt_s[1:2, NGO + 2 * NKE:NGO + 3 * NKE]

        kest = k0 * sc(14) + k1 * sc(15)
        wspke = wsp_s[:, G + NGO:G + NGO + NKE]
        sp_ke = _dotT(wspke, kest) + sc(16) * jnp.sum(wspke)
        sp_gene = (sp_ref[:, 0:32] * sc(20) + sp_ref[:, 32:64] * sc(21)
                   + sp_ref[:, 64:65] * sc(22))
        state_ref[...] = sp_gene + sp_go + sp_ke + sc(23)

        kelay = k0 * sc(17) + k1 * sc(18) + sc(19)
        bio_bt = jnp.dot(kelay, m_s[...], preferred_element_type=F32)
        xb = _dotT(bio_bt, wb_s[...]) + bbio_s[...]
        yb = _fourier32(xb, p_s[:, 0:64], b1b_s[...])

        xd = _dotT(comp_s[...], wdrug_s[...]) + bdrug_s[...]
        yd = _fourier32(xd, p_s[:, 64:128], d1b_s[...])

        comb = jnp.concatenate([yb, yd], axis=1)
        yp = (_fourier32(comb, p_s[:, 128:132], jnp.zeros((1, 1), F32))
              + sc(24))
        pred_ref[...] = yp


def kernel(gene, gene_go, go_ke, ke_ke, tissue, compound, W_gene1, b_gene1,
           bn_gamma, bn_beta, W_gstate, b_gstate, g2g_coeffs, g2g_nbias,
           W_goenc, b_goenc, W_godec, b_godec, W_gostate, b_gostate,
           g2k_coeffs, g2k_nbias, k2k0_coeffs, k2k0_nbias, k2k1_coeffs,
           k2k1_nbias, W_kestate, b_kestate, W_kelayer, b_kelayer, W_sp, b_sp,
           W_bio0, b_bio0, bio1_coeffs, bio1_bias, W_drug0, b_drug0,
           drug1_coeffs, drug1_bias, pred_coeffs, pred_bias):
    nbcat = jnp.concatenate([g2g_nbias, g2k_nbias, k2k0_nbias, k2k1_nbias],
                            axis=0)
    cs = jnp.concatenate([
        jnp.stack([
            W_gene1[0, 0], W_gene1[1, 0], b_gene1[0], b_gene1[1],
            W_goenc[0, 0], W_goenc[0, 1], b_goenc[0],
            W_godec[0, 0], W_godec[1, 0], b_godec[0], b_godec[1],
            W_gostate[0, 0], W_gostate[0, 1], b_gostate[0],
            W_kestate[0, 0], W_kestate[0, 1], b_kestate[0],
            W_kelayer[0, 0], W_kelayer[0, 1], b_kelayer[0],
            W_gstate[0, 0], W_gstate[0, 1], b_gstate[0],
            b_sp[0], pred_bias[0, 0], 0.0, 0.0, 0.0, 0.0, 0.0, 0.0, 0.0]),
        g2g_coeffs.reshape(16), g2k_coeffs.reshape(16),
        k2k0_coeffs.reshape(16), k2k1_coeffs.reshape(16)]).reshape(1, 96)

    sm = pl.BlockSpec(memory_space=pltpu.MemorySpace.SMEM)
    anys = pl.BlockSpec(memory_space=pl.ANY)

    pred, state_row = pl.pallas_call(
        _body,
        grid=(NSTEP,),
        in_specs=[anys, pl.BlockSpec((MBLK, G), lambda i: (i, 0))]
        + [anys] * 17 + [sm],
        out_specs=[
            pl.BlockSpec((B, 1), lambda i: (0, 0)),
            pl.BlockSpec((1, B), lambda i: (0, 0)),
        ],
        out_shape=[
            jax.ShapeDtypeStruct((B, 1), F32),
            jax.ShapeDtypeStruct((1, B), F32),
        ],
        scratch_shapes=[
            pltpu.VMEM((B + 2, G), F32),
            pltpu.VMEM((1, G + NGO + NKE), F32),
            pltpu.VMEM((G, 128), BF16),
            pltpu.VMEM((NGO, 128), F32),
            pltpu.VMEM((NKE, NGO), F32),
            pltpu.VMEM((NKE, NKE), F32),
            pltpu.VMEM((NBALL, 2), F32),
            pltpu.VMEM((B, DC), F32),
            pltpu.VMEM((B, DC), F32),
            pltpu.VMEM((B, NT), F32),
            pltpu.VMEM((1, NT), jnp.int32),
            pltpu.VMEM((32, 64), F32),
            pltpu.VMEM((32, 64), F32),
            pltpu.VMEM((2, 64), F32),
            pltpu.VMEM((1, B), F32),
            pltpu.VMEM((1, B), F32),
            pltpu.VMEM((1, 16), F32),
            pltpu.VMEM((1, 16), F32),
            pltpu.VMEM((2, NBALL), F32),
            pltpu.VMEM((B, 132), F32),
            pltpu.VMEM((NKE, 64), F32),
            pltpu.VMEM((B, 64), F32),
            pltpu.VMEM((1, 1024), F32),
            pltpu.VMEM((1, 128), F32),
            pltpu.SemaphoreType.DMA((18,)),
        ],
    )(gene, gene_go, bn_gamma.reshape(1, G), bn_beta.reshape(1, G), W_sp,
      go_ke, ke_ke, nbcat, compound, W_drug0, W_bio0,
      tissue.astype(jnp.int32).reshape(1, NT),
      bio1_coeffs.reshape(32, 64), drug1_coeffs.reshape(32, 64),
      pred_coeffs.reshape(2, 64), b_bio0.reshape(1, B), b_drug0.reshape(1, B),
      bio1_bias, drug1_bias, cs)

    return pred, state_row.reshape(B, 1)


# speedup vs baseline: 1.0176x; 1.0176x over previous
import jax
import jax.numpy as jnp
from jax.experimental import pallas as pl
from jax.experimental.pallas import tpu as pltpu

B, G, NGO, NKE, NN, GRID, NT, DC = 32, 10000, 2000, 500, 2, 2, 50, 256
F32 = jnp.float32
BF16 = jnp.bfloat16
MBLK = 200
NSTEP = NGO // MBLK
NBALL = NGO + 3 * NKE


def _dotT(a, b):
    return jax.lax.dot_general(a, b, (((1,), (1,)), ((), ())),
                               preferred_element_type=F32)


def _fk2(x0, x1, cs_ref, base):
    def c(ci, j, i, g):
        return cs_ref[0, base + ci * 8 + j * 4 + i * 2 + g]
    c10, s10 = jnp.cos(x0), jnp.sin(x0)
    c11, s11 = jnp.cos(x1), jnp.sin(x1)
    c20 = 2.0 * c10 * c10 - 1.0
    s20 = 2.0 * s10 * c10
    c21 = 2.0 * c11 * c11 - 1.0
    s21 = 2.0 * s11 * c11
    y0 = (c10 * c(0, 0, 0, 0) + c20 * c(0, 0, 0, 1) + c11 * c(0, 0, 1, 0)
          + c21 * c(0, 0, 1, 1) + s10 * c(1, 0, 0, 0) + s20 * c(1, 0, 0, 1)
          + s11 * c(1, 0, 1, 0) + s21 * c(1, 0, 1, 1))
    y1 = (c10 * c(0, 1, 0, 0) + c20 * c(0, 1, 0, 1) + c11 * c(0, 1, 1, 0)
          + c21 * c(0, 1, 1, 1) + s10 * c(1, 1, 0, 0) + s20 * c(1, 1, 0, 1)
          + s11 * c(1, 1, 1, 0) + s21 * c(1, 1, 1, 1))
    return y0, y1


def _fourier32(x, p, bias):
    j = p.shape[1] // 4
    cx, sx = jnp.cos(x), jnp.sin(x)
    c2, s2 = 2.0 * cx * cx - 1.0, 2.0 * sx * cx
    y = (jnp.dot(cx, p[:, 0:j], preferred_element_type=F32)
         + jnp.dot(c2, p[:, j:2 * j], preferred_element_type=F32)
         + jnp.dot(sx, p[:, 2 * j:3 * j], preferred_element_type=F32)
         + jnp.dot(s2, p[:, 3 * j:4 * j], preferred_element_type=F32))
    return y + bias


def _body(gene_hbm, adj_ref, gam_hbm, bet_hbm, wsp_hbm, goke_hbm, keke_hbm,
          nbc_hbm, comp_hbm, wdrug_hbm, wbio_hbm, tis_hbm, cb_hbm, cd_hbm,
          cp_hbm, bbio_hbm, bdrug_hbm, b1b_hbm, d1b_hbm, cs_ref,
          pred_ref, state_ref,
          gp_s, wsp_s, zt_ref, h_ref, goke_s, keke_s, nbc_s, comp_s, wdrug_s,
          wbio_s, tis_s, cb_s, cd_s, cp_s, bbio_s, bdrug_s, b1b_s, d1b_s,
          nbt_s, p_s, m_s, wb_s, dk_s, sp_ref, sem):
    i = pl.program_id(0)
    sc = lambda k: cs_ref[0, k]
    copies = [
        (gene_hbm, gp_s.at[0:B, :]), (gam_hbm, gp_s.at[B:B + 1, :]),
        (bet_hbm, gp_s.at[B + 1:B + 2, :]), (wsp_hbm, wsp_s),
        (goke_hbm, goke_s), (keke_hbm, keke_s), (nbc_hbm, nbc_s),
        (comp_hbm, comp_s), (wdrug_hbm, wdrug_s), (wbio_hbm, wbio_s),
        (tis_hbm, tis_s), (cb_hbm, cb_s), (cd_hbm, cd_s), (cp_hbm, cp_s),
        (bbio_hbm, bbio_s), (bdrug_hbm, bdrug_s), (b1b_hbm, b1b_s),
        (d1b_hbm, d1b_s),
    ]

    @pl.when(i == 0)
    def _build_zt():
        for k, (src, dst) in enumerate(copies):
            pltpu.make_async_copy(src, dst, sem.at[k]).start()
        for k in range(4):
            src, dst = copies[k]
            pltpu.make_async_copy(src, dst, sem.at[k]).wait()
        g = gp_s[0:B, :]
        t0 = jnp.tanh(g * sc(0) + sc(2))
        t1 = jnp.tanh(g * sc(1) + sc(3))
        s1 = jnp.sum(t0, axis=0, keepdims=True) + jnp.sum(t1, axis=0, keepdims=True)
        s2 = jnp.sum(t0 * t0, axis=0, keepdims=True) + jnp.sum(t1 * t1, axis=0, keepdims=True)
        mean = s1 * (1.0 / (2 * B))
        var = s2 * (1.0 / (2 * B)) - mean * mean
        inv = jax.lax.rsqrt(var + 1e-5) * gp_s[B:B + 1, :]
        bet = gp_s[B + 1:B + 2, :]
        zn0 = ((t0 - mean) * inv + bet).astype(BF16)
        zn1 = ((t1 - mean) * inv + bet).astype(BF16)
        zt_ref[...] = jnp.zeros((G, 128), BF16)
        zt_ref[:, 0:32] = jnp.transpose(zn0)
        zt_ref[:, 32:64] = jnp.transpose(zn1)
        zt_ref[:, 64:65] = jnp.ones((G, 1), BF16)
        sp_ref[...] = jnp.dot(wsp_s[:, 0:G].astype(BF16), zt_ref[...],
                              preferred_element_type=F32)

    h_ref[pl.ds(i * MBLK, MBLK), :] = jnp.dot(
        adj_ref[...].astype(BF16), zt_ref[...], preferred_element_type=F32)

    @pl.when(i == 1)
    def _prep_small():
        for k in range(6, len(copies)):
            src, dst = copies[k]
            pltpu.make_async_copy(src, dst, sem.at[k]).wait()
        nbt_s[...] = jnp.transpose(nbc_s[...])
        li = jax.lax.broadcasted_iota(jnp.int32, (B, 64), 1)
        si = jax.lax.broadcasted_iota(jnp.int32, (B, 64), 0)
        e0 = (li == 2 * si).astype(F32)
        e1 = (li == 2 * si + 1).astype(F32)
        p_s[:, 0:16] = _dotT(e0, cb_s[0:16, :])
        p_s[:, 16:32] = _dotT(e1, cb_s[0:16, :])
        p_s[:, 32:48] = _dotT(e0, cb_s[16:32, :])
        p_s[:, 48:64] = _dotT(e1, cb_s[16:32, :])
        p_s[:, 64:80] = _dotT(e0, cd_s[0:16, :])
        p_s[:, 80:96] = _dotT(e1, cd_s[0:16, :])
        p_s[:, 96:112] = _dotT(e0, cd_s[16:32, :])
        p_s[:, 112:128] = _dotT(e1, cd_s[16:32, :])
        p_s[:, 128:129] = _dotT(e0, cp_s[0:1, :])
        p_s[:, 129:130] = _dotT(e1, cp_s[0:1, :])
        p_s[:, 130:131] = _dotT(e0, cp_s[1:2, :])
        p_s[:, 131:132] = _dotT(e1, cp_s[1:2, :])
        tis64 = jnp.concatenate(
            [tis_s[...], jnp.full((1, 64 - NT), -1, jnp.int32)], axis=1)
        kidx = jax.lax.broadcasted_iota(jnp.int32, (NKE, 64), 0)
        m_s[...] = (kidx == tis64).astype(F32)
        wb_s[...] = jnp.concatenate(
            [wbio_s[...], jnp.zeros((B, 64 - NT), F32)], axis=1)

    @pl.when(i == 3)
    def _prep_deg():
        for k in (4, 5):
            src, dst = copies[k]
            pltpu.make_async_copy(src, dst, sem.at[k]).wait()
        dk_s[:, 0:NKE] = _dotT(jnp.ones((1, NGO), F32), goke_s[...]) + 1e-8
        dk_s[:, 512:512 + NKE] = (_dotT(jnp.ones((1, NKE), F32), keke_s[...])
                                  + 1e-8)

    @pl.when(i == NSTEP - 1)
    def _tail():
        ht = jnp.transpose(h_ref[...])
        degT = ht[64:65, :] + 1e-8
        x0 = ht[0:32, :] / degT
        x1 = ht[32:64, :] / degT
        y0, y1 = _fk2(x0, x1, cs_ref, 32)
        y0 = y0 + nbt_s[0:1, 0:NGO]
        y1 = y1 + nbt_s[1:2, 0:NGO]
        e = y0 * sc(4) + y1 * sc(5) + sc(6)
        gf0 = jnp.tanh(e * sc(7) + sc(9))
        gf1 = jnp.tanh(e * sc(8) + sc(10))
        gost = gf0 * sc(11) + gf1 * sc(12)
        wspgo = wsp_s[:, G:G + NGO]
        sp_go = _dotT(wspgo, gost) + sc(13) * jnp.sum(wspgo)

        a = goke_s[...]
        dkg = dk_s[:, 0:NKE]
        kh0 = _dotT(gf0, a) / dkg
        kh1 = _dotT(gf1, a) / dkg
        y0, y1 = _fk2(kh0, kh1, cs_ref, 48)
        k0 = y0 + nbt_s[0:1, NGO:NGO + NKE]
        k1 = y1 + nbt_s[1:2, NGO:NGO + NKE]

        kk = keke_s[...]
        dkk = dk_s[:, 512:512 + NKE]
        x0 = _dotT(k0, kk) / dkk
        x1 = _dotT(k1, kk) / dkk
        y0, y1 = _fk2(x0, x1, cs_ref, 64)
        k0 = y0 + nbt_s[0:1, NGO + NKE:NGO + 2 * NKE]
        k1 = y1 + nbt_s[1:2, NGO + NKE:NGO + 2 * NKE]
        x0 = _dotT(k0, kk) / dkk
        x1 = _dotT(k1, kk) / dkk
        y0, y1 = _fk2(x0, x1, cs_ref, 80)
        k0 = y0 + nbt_s[0:1, NGO + 2 * NKE:NGO + 3 * NKE]
        k1 = y1 + nbt_s[1:2, NGO + 2 * NKE:NGO + 3 * NKE]

        kest = k0 * sc(14) + k1 * sc(15)
        wspke = wsp_s[:, G + NGO:G + NGO + NKE]
        sp_ke = _dotT(wspke, kest) + sc(16) * jnp.sum(wspke)
        sp_gene = (sp_ref[:, 0:32] * sc(20) + sp_ref[:, 32:64] * sc(21)
                   + sp_ref[:, 64:65] * sc(22))
        state_ref[...] = sp_gene + sp_go + sp_ke + sc(23)

        kelay = k0 * sc(17) + k1 * sc(18) + sc(19)
        bio_bt = jnp.dot(kelay, m_s[...], preferred_element_type=F32)
        xb = _dotT(bio_bt, wb_s[...]) + bbio_s[...]
        yb = _fourier32(xb, p_s[:, 0:64], b1b_s[...])

        xd = _dotT(comp_s[...], wdrug_s[...]) + bdrug_s[...]
        yd = _fourier32(xd, p_s[:, 64:128], d1b_s[...])

        comb = jnp.concatenate([yb, yd], axis=1)
        yp = (_fourier32(comb, p_s[:, 128:132], jnp.zeros((1, 1), F32))
              + sc(24))
        pred_ref[...] = yp


def kernel(gene, gene_go, go_ke, ke_ke, tissue, compound, W_gene1, b_gene1,
           bn_gamma, bn_beta, W_gstate, b_gstate, g2g_coeffs, g2g_nbias,
           W_goenc, b_goenc, W_godec, b_godec, W_gostate, b_gostate,
           g2k_coeffs, g2k_nbias, k2k0_coeffs, k2k0_nbias, k2k1_coeffs,
           k2k1_nbias, W_kestate, b_kestate, W_kelayer, b_kelayer, W_sp, b_sp,
           W_bio0, b_bio0, bio1_coeffs, bio1_bias, W_drug0, b_drug0,
           drug1_coeffs, drug1_bias, pred_coeffs, pred_bias):
    nbcat = jnp.concatenate([g2g_nbias, g2k_nbias, k2k0_nbias, k2k1_nbias],
                            axis=0)
    cs = jnp.concatenate([
        jnp.stack([
            W_gene1[0, 0], W_gene1[1, 0], b_gene1[0], b_gene1[1],
            W_goenc[0, 0], W_goenc[0, 1], b_goenc[0],
            W_godec[0, 0], W_godec[1, 0], b_godec[0], b_godec[1],
            W_gostate[0, 0], W_gostate[0, 1], b_gostate[0],
            W_kestate[0, 0], W_kestate[0, 1], b_kestate[0],
            W_kelayer[0, 0], W_kelayer[0, 1], b_kelayer[0],
            W_gstate[0, 0], W_gstate[0, 1], b_gstate[0],
            b_sp[0], pred_bias[0, 0], 0.0, 0.0, 0.0, 0.0, 0.0, 0.0, 0.0]),
        g2g_coeffs.reshape(16), g2k_coeffs.reshape(16),
        k2k0_coeffs.reshape(16), k2k1_coeffs.reshape(16)]).reshape(1, 96)

    sm = pl.BlockSpec(memory_space=pltpu.MemorySpace.SMEM)
    anys = pl.BlockSpec(memory_space=pl.ANY)

    pred, state_row = pl.pallas_call(
        _body,
        grid=(NSTEP,),
        in_specs=[anys, pl.BlockSpec((MBLK, G), lambda i: (i, 0))]
        + [anys] * 17 + [sm],
        out_specs=[
            pl.BlockSpec((B, 1), lambda i: (0, 0)),
            pl.BlockSpec((1, B), lambda i: (0, 0)),
        ],
        out_shape=[
            jax.ShapeDtypeStruct((B, 1), F32),
            jax.ShapeDtypeStruct((1, B), F32),
        ],
        scratch_shapes=[
            pltpu.VMEM((B + 2, G), F32),
            pltpu.VMEM((1, G + NGO + NKE), F32),
            pltpu.VMEM((G, 128), BF16),
            pltpu.VMEM((NGO, 128), F32),
            pltpu.VMEM((NKE, NGO), F32),
            pltpu.VMEM((NKE, NKE), F32),
            pltpu.VMEM((NBALL, 2), F32),
            pltpu.VMEM((B, DC), F32),
            pltpu.VMEM((B, DC), F32),
            pltpu.VMEM((B, NT), F32),
            pltpu.VMEM((1, NT), jnp.int32),
            pltpu.VMEM((32, 64), F32),
            pltpu.VMEM((32, 64), F32),
            pltpu.VMEM((2, 64), F32),
            pltpu.VMEM((1, B), F32),
            pltpu.VMEM((1, B), F32),
            pltpu.VMEM((1, 16), F32),
            pltpu.VMEM((1, 16), F32),
            pltpu.VMEM((2, NBALL), F32),
            pltpu.VMEM((B, 132), F32),
            pltpu.VMEM((NKE, 64), F32),
            pltpu.VMEM((B, 64), F32),
            pltpu.VMEM((1, 1024), F32),
            pltpu.VMEM((1, 128), F32),
            pltpu.SemaphoreType.DMA((18,)),
        ],
    )(gene, gene_go, bn_gamma.reshape(1, G), bn_beta.reshape(1, G), W_sp,
      go_ke, ke_ke, nbcat, compound, W_drug0, W_bio0,
      tissue.astype(jnp.int32).reshape(1, NT),
      bio1_coeffs.reshape(32, 64), drug1_coeffs.reshape(32, 64),
      pred_coeffs.reshape(2, 64), b_bio0.reshape(1, B), b_drug0.reshape(1, B),
      bio1_bias, drug1_bias, cs)

    return pred, state_row.reshape(B, 1)
